# baseline (speedup 1.0000x reference)
import functools

import jax
import jax.numpy as jnp
from jax.experimental import pallas as pl
from jax.experimental.pallas import tpu as pltpu


def _adjacency_block(refs_blk_ref, refs_ref, srcs_ref, out_ref, *, block_n):
    i = pl.program_id(1)

    refs = refs_ref[0]
    refs_blk = refs_blk_ref[0]
    sq = jnp.sum(refs * refs, axis=0, keepdims=True)
    e = jnp.exp(-sq)
    ssrcs = srcs_ref[0] * e
    e_blk = jnp.exp(-jnp.sum(refs_blk * refs_blk, axis=0,
                             keepdims=True))

    inner2 = jax.lax.dot_general(
        2.0 * refs_blk, refs,
        dimension_numbers=(((0,), (0,)), ((), ())),
        preferred_element_type=jnp.float32,
    )
    w = jnp.exp(inner2).astype(jnp.bfloat16)

    filt = jax.lax.dot_general(
        ssrcs.astype(jnp.bfloat16), w,
        dimension_numbers=(((1,), (1,)), ((), ())),
        preferred_element_type=jnp.float32,
    )
    out_ref[0] = filt * e_blk - srcs_ref[0, :, pl.ds(i * block_n, block_n)]


def kernel(src_imgs, guide_imgs):
    bs, L, h, w = src_imgs.shape
    d = guide_imgs.shape[1]
    n = h * w

    flat_srcs = src_imgs.reshape(bs, L, n)
    flat_refs = guide_imgs.reshape(bs, d, n)

    block_n = 1024
    grid = (bs, n // block_n)

    out = pl.pallas_call(
        functools.partial(_adjacency_block, block_n=block_n),
        grid=grid,
        in_specs=[
            pl.BlockSpec((1, d, block_n), lambda b, i: (b, 0, i)),
            pl.BlockSpec((1, d, n), lambda b, i: (b, 0, 0)),
            pl.BlockSpec((1, L, n), lambda b, i: (b, 0, 0)),
        ],
        out_specs=pl.BlockSpec((1, L, block_n), lambda b, i: (b, 0, i)),
        out_shape=jax.ShapeDtypeStruct((bs, L, n), jnp.float32),
        compiler_params=pltpu.CompilerParams(
            dimension_semantics=("parallel", "parallel")),
    )(flat_refs, flat_refs, flat_srcs)

    return out.reshape(bs, L, h, w)


# speedup vs baseline: 1.0353x; 1.0353x over previous
import functools

import jax
import jax.numpy as jnp
from jax.experimental import pallas as pl
from jax.experimental.pallas import tpu as pltpu


def _adjacency_block(refs_blk_ref, refs_ref, srcs_ref, out_ref, *, block_n,
                     tile_j, n):
    i = pl.program_id(1)

    refs_blk2 = 2.0 * refs_blk_ref[0]
    e_blk = jnp.exp(-jnp.sum(refs_blk_ref[0] * refs_blk_ref[0], axis=0,
                             keepdims=True))

    filt = None
    for jt in range(n // tile_j):
        refs_t = refs_ref[0, :, pl.ds(jt * tile_j, tile_j)]
        srcs_t = srcs_ref[0, :, pl.ds(jt * tile_j, tile_j)]
        sq_t = jnp.sum(refs_t * refs_t, axis=0, keepdims=True)
        ssrcs_t = (srcs_t * jnp.exp(-sq_t)).astype(jnp.bfloat16)

        inner2 = jax.lax.dot_general(
            refs_blk2, refs_t,
            dimension_numbers=(((0,), (0,)), ((), ())),
            preferred_element_type=jnp.float32,
        )
        w = jnp.exp(inner2).astype(jnp.bfloat16)

        part = jax.lax.dot_general(
            ssrcs_t, w,
            dimension_numbers=(((1,), (1,)), ((), ())),
            preferred_element_type=jnp.float32,
        )
        filt = part if filt is None else filt + part

    out_ref[0] = filt * e_blk - srcs_ref[0, :, pl.ds(i * block_n, block_n)]


def kernel(src_imgs, guide_imgs):
    bs, L, h, w = src_imgs.shape
    d = guide_imgs.shape[1]
    n = h * w

    flat_srcs = src_imgs.reshape(bs, L, n)
    flat_refs = guide_imgs.reshape(bs, d, n)

    block_n = 1024
    tile_j = 1024
    grid = (bs, n // block_n)

    out = pl.pallas_call(
        functools.partial(_adjacency_block, block_n=block_n, tile_j=tile_j,
                          n=n),
        grid=grid,
        in_specs=[
            pl.BlockSpec((1, d, block_n), lambda b, i: (b, 0, i)),
            pl.BlockSpec((1, d, n), lambda b, i: (b, 0, 0)),
            pl.BlockSpec((1, L, n), lambda b, i: (b, 0, 0)),
        ],
        out_specs=pl.BlockSpec((1, L, block_n), lambda b, i: (b, 0, i)),
        out_shape=jax.ShapeDtypeStruct((bs, L, n), jnp.float32),
        compiler_params=pltpu.CompilerParams(
            dimension_semantics=("parallel", "parallel")),
    )(flat_refs, flat_refs, flat_srcs)

    return out.reshape(bs, L, h, w)


# speedup vs baseline: 1.0446x; 1.0089x over previous
import functools

import jax
import jax.numpy as jnp
from jax.experimental import pallas as pl
from jax.experimental.pallas import tpu as pltpu


def _adjacency_block(refs_blk_ref, refs_ref, srcs_ref, out_ref, *, block_n,
                     tile_j, n):
    i = pl.program_id(1)

    refs_blk2 = 2.0 * refs_blk_ref[0]
    e_blk = jnp.exp(-jnp.sum(refs_blk_ref[0] * refs_blk_ref[0], axis=0,
                             keepdims=True))

    filt = None
    for jt in range(n // tile_j):
        refs_t = refs_ref[0, :, pl.ds(jt * tile_j, tile_j)]
        srcs_t = srcs_ref[0, :, pl.ds(jt * tile_j, tile_j)]
        sq_t = jnp.sum(refs_t * refs_t, axis=0, keepdims=True)
        ssrcs_t = (srcs_t * jnp.exp(-sq_t)).astype(jnp.bfloat16)

        inner2 = jax.lax.dot_general(
            refs_blk2, refs_t,
            dimension_numbers=(((0,), (0,)), ((), ())),
            preferred_element_type=jnp.float32,
        )
        w = jnp.exp(inner2).astype(jnp.bfloat16)

        part = jax.lax.dot_general(
            ssrcs_t, w,
            dimension_numbers=(((1,), (1,)), ((), ())),
            preferred_element_type=jnp.float32,
        )
        filt = part if filt is None else filt + part

    out_ref[0] = filt * e_blk - srcs_ref[0, :, pl.ds(i * block_n, block_n)]


def kernel(src_imgs, guide_imgs):
    bs, L, h, w = src_imgs.shape
    d = guide_imgs.shape[1]
    n = h * w

    flat_srcs = src_imgs.reshape(bs, L, n)
    flat_refs = guide_imgs.reshape(bs, d, n)

    block_n = 1024
    tile_j = 512
    grid = (bs, n // block_n)

    out = pl.pallas_call(
        functools.partial(_adjacency_block, block_n=block_n, tile_j=tile_j,
                          n=n),
        grid=grid,
        in_specs=[
            pl.BlockSpec((1, d, block_n), lambda b, i: (b, 0, i)),
            pl.BlockSpec((1, d, n), lambda b, i: (b, 0, 0)),
            pl.BlockSpec((1, L, n), lambda b, i: (b, 0, 0)),
        ],
        out_specs=pl.BlockSpec((1, L, block_n), lambda b, i: (b, 0, i)),
        out_shape=jax.ShapeDtypeStruct((bs, L, n), jnp.float32),
        compiler_params=pltpu.CompilerParams(
            dimension_semantics=("parallel", "parallel")),
    )(flat_refs, flat_refs, flat_srcs)

    return out.reshape(bs, L, h, w)
